# baseline (speedup 1.0000x reference)
import functools

import jax
import jax.numpy as jnp
from jax import lax
from jax.experimental import pallas as pl
from jax.experimental.pallas import tpu as pltpu
from jax.experimental.pallas import tpu_sc as plsc

_L = 16


def _deg_partials_call(dst, n_pad, num_workers, nc):
    e_total = dst.shape[0]
    e_per_w = e_total // num_workers
    mesh = plsc.VectorSubcoreMesh(core_axis_name="c", subcore_axis_name="s")

    @functools.partial(
        pl.kernel,
        mesh=mesh,
        out_type=jax.ShapeDtypeStruct((num_workers, n_pad), jnp.float32),
        scratch_types=[
            pltpu.VMEM((e_per_w,), jnp.int32),
            pltpu.VMEM((n_pad,), jnp.float32),
        ],
        compiler_params=pltpu.CompilerParams(needs_layout_passes=False),
    )
    def deg_kernel(dst_hbm, out_hbm, idx_v, bins_v):
        wid = lax.axis_index("s") * nc + lax.axis_index("c")
        base = wid * e_per_w
        pltpu.sync_copy(dst_hbm.at[pl.ds(base, e_per_w)], idx_v)

        zeros = jnp.zeros((_L,), jnp.float32)
        ones = jnp.ones((_L,), jnp.float32)

        @plsc.parallel_loop(0, n_pad // _L, unroll=8)
        def _zero(i):
            bins_v[pl.ds(pl.multiple_of(i * _L, _L), _L)] = zeros

        @plsc.parallel_loop(0, e_per_w // _L, unroll=8)
        def _scat(i):
            idx = idx_v[pl.ds(pl.multiple_of(i * _L, _L), _L)]
            plsc.addupdate_scatter(bins_v, [idx], ones)
        pltpu.sync_copy(bins_v, out_hbm.at[wid])

    return deg_kernel(dst)


def _dinv_call(deg_partials):
    def body(p_ref, o_ref):
        deg = 1.0 + jnp.sum(p_ref[...], axis=0, keepdims=True)
        o_ref[...] = 1.0 / jnp.sqrt(deg)

    n_pad = deg_partials.shape[1]
    return pl.pallas_call(
        body,
        out_shape=jax.ShapeDtypeStruct((1, n_pad), jnp.float32),
    )(deg_partials)


def _w_partials_call(src, dst, dinv, n_pad, num_workers, nc):
    e_total = src.shape[0]
    e_per_w = e_total // num_workers
    mesh = plsc.VectorSubcoreMesh(core_axis_name="c", subcore_axis_name="s")

    @functools.partial(
        pl.kernel,
        mesh=mesh,
        out_type=jax.ShapeDtypeStruct((num_workers, n_pad), jnp.float32),
        scratch_types=[
            pltpu.VMEM((e_per_w,), jnp.int32),
            pltpu.VMEM((e_per_w,), jnp.int32),
            pltpu.VMEM((n_pad,), jnp.float32),
            pltpu.VMEM((n_pad,), jnp.float32),
        ],
        compiler_params=pltpu.CompilerParams(needs_layout_passes=False),
    )
    def w_kernel(src_hbm, dst_hbm, dinv_hbm, out_hbm, src_v, dst_v, dinv_v, bins_v):
        wid = lax.axis_index("s") * nc + lax.axis_index("c")
        base = wid * e_per_w
        pltpu.sync_copy(src_hbm.at[pl.ds(base, e_per_w)], src_v)
        pltpu.sync_copy(dst_hbm.at[pl.ds(base, e_per_w)], dst_v)
        pltpu.sync_copy(dinv_hbm, dinv_v)

        is_w0 = wid == 0
        zeros = jnp.zeros((_L,), jnp.float32)

        @plsc.parallel_loop(0, n_pad // _L, unroll=8)
        def _init(i):
            sl = pl.ds(pl.multiple_of(i * _L, _L), _L)
            dv = dinv_v[sl]
            bins_v[sl] = jnp.where(is_w0, dv * dv, zeros)

        @plsc.parallel_loop(0, e_per_w // _L, unroll=8)
        def _scat(i):
            sl = pl.ds(pl.multiple_of(i * _L, _L), _L)
            s = src_v[sl]
            d = dst_v[sl]
            a = plsc.load_gather(dinv_v, [s])
            b = plsc.load_gather(dinv_v, [d])
            plsc.addupdate_scatter(bins_v, [s], a * b)
        pltpu.sync_copy(bins_v, out_hbm.at[wid])

    return w_kernel(src, dst, dinv.reshape(-1))


def _pool_lstm_call(xr, w_partials, W_gcn, b_gcn, W_ihT, W_hhT, b_ih, b_hh,
                    W_clsT, b_cls, Bsz, Tlen):
    BT, Nn, Fin = xr.shape
    H = W_hhT.shape[0]
    O = W_clsT.shape[1]
    num_w = w_partials.shape[0]
    CN = 2000
    grid = Nn // CN
    inv_n = 1.0 / Nn
    wp3 = w_partials[:, :Nn].reshape(num_w, grid, 1, CN)

    def body(x_ref, wp_ref, wg_ref, bg_ref, wih_ref, whh_ref, bih_ref,
             bhh_ref, wcls_ref, bcls_ref, o_ref, acc_ref):
        i = pl.program_id(0)

        @pl.when(i == 0)
        def _():
            acc_ref[...] = jnp.zeros_like(acc_ref)

        w_chunk = jnp.sum(wp_ref[..., 0, :], axis=0)
        x = x_ref[...]
        BTl, CNl, Fl = x.shape
        xl = jnp.dot(x.reshape(BTl * CNl, Fl), wg_ref[...],
                     preferred_element_type=jnp.float32)
        xl = xl.reshape(BTl, CNl, xl.shape[-1])
        acc_ref[...] += jnp.sum(xl * w_chunk[:, :, None], axis=1)

        @pl.when(i == grid - 1)
        def _():
            pooled = acc_ref[...] * inv_n + bg_ref[...]
            h = jnp.zeros((Bsz, H), jnp.float32)
            c = jnp.zeros((Bsz, H), jnp.float32)
            b_gates = bih_ref[...] + bhh_ref[...]
            for t in range(Tlen):
                xt = jnp.concatenate(
                    [pooled[b * Tlen + t:b * Tlen + t + 1] for b in range(Bsz)],
                    axis=0)
                gates = (jnp.dot(xt, wih_ref[...],
                                 preferred_element_type=jnp.float32)
                         + jnp.dot(h, whh_ref[...],
                                   preferred_element_type=jnp.float32)
                         + b_gates)
                i_g = jax.nn.sigmoid(gates[:, 0:H])
                f_g = jax.nn.sigmoid(gates[:, H:2 * H])
                g_g = jnp.tanh(gates[:, 2 * H:3 * H])
                o_g = jax.nn.sigmoid(gates[:, 3 * H:4 * H])
                c = f_g * c + i_g * g_g
                h = o_g * jnp.tanh(c)
            o_ref[...] = jnp.dot(h, wcls_ref[...],
                                 preferred_element_type=jnp.float32) + bcls_ref[...]

    full = lambda shape: pl.BlockSpec(shape, lambda i: tuple(0 for _ in shape))
    return pl.pallas_call(
        body,
        grid=(grid,),
        in_specs=[
            pl.BlockSpec((BT, CN, Fin), lambda i: (0, i, 0)),
            pl.BlockSpec((num_w, 1, 1, CN), lambda i: (0, i, 0, 0)),
            full(W_gcn.shape),
            full(b_gcn.shape),
            full(W_ihT.shape),
            full(W_hhT.shape),
            full(b_ih.shape),
            full(b_hh.shape),
            full(W_clsT.shape),
            full(b_cls.shape),
        ],
        out_specs=pl.BlockSpec((Bsz, O), lambda i: (0, 0)),
        out_shape=jax.ShapeDtypeStruct((Bsz, O), jnp.float32),
        scratch_shapes=[pltpu.VMEM((BT, Fin), jnp.float32)],
        compiler_params=pltpu.CompilerParams(
            dimension_semantics=("arbitrary",)),
    )(xr, wp3, W_gcn, b_gcn, W_ihT, W_hhT, b_ih, b_hh, W_clsT, b_cls)


def kernel(x_seq, edge_index, W_gcn, b_gcn, W_ih, W_hh, b_ih, b_hh, W_cls, b_cls):
    Bsz, Tlen, Nn, Fin = x_seq.shape
    info = plsc.get_sparse_core_info()
    nc, ns = info.num_cores, info.num_subcores
    num_workers = nc * ns

    n_pad = ((Nn + 127) // 128) * 128
    src = edge_index[0]
    dst = edge_index[1]

    deg_partials = _deg_partials_call(dst, n_pad, num_workers, nc)
    dinv = _dinv_call(deg_partials)
    w_partials = _w_partials_call(src, dst, dinv, n_pad, num_workers, nc)

    xr = x_seq.reshape(Bsz * Tlen, Nn, Fin)
    logits = _pool_lstm_call(
        xr, w_partials, W_gcn, b_gcn.reshape(1, -1), W_ih.T, W_hh.T,
        b_ih.reshape(1, -1), b_hh.reshape(1, -1), W_cls.T,
        b_cls.reshape(1, -1), Bsz, Tlen)
    return logits


# speedup vs baseline: 1.0146x; 1.0146x over previous
import functools

import jax
import jax.numpy as jnp
from jax import lax
from jax.experimental import pallas as pl
from jax.experimental.pallas import tpu as pltpu
from jax.experimental.pallas import tpu_sc as plsc

_L = 16


def _deg_partials_call(dst, n_pad, num_workers, nc):
    e_total = dst.shape[0]
    e_per_w = e_total // num_workers
    mesh = plsc.VectorSubcoreMesh(core_axis_name="c", subcore_axis_name="s")

    @functools.partial(
        pl.kernel,
        mesh=mesh,
        out_type=jax.ShapeDtypeStruct((num_workers, n_pad), jnp.float32),
        scratch_types=[
            pltpu.VMEM((e_per_w,), jnp.int32),
            pltpu.VMEM((n_pad,), jnp.float32),
        ],
        compiler_params=pltpu.CompilerParams(needs_layout_passes=False),
    )
    def deg_kernel(dst_hbm, out_hbm, idx_v, bins_v):
        wid = lax.axis_index("s") * nc + lax.axis_index("c")
        base = wid * e_per_w
        pltpu.sync_copy(dst_hbm.at[pl.ds(base, e_per_w)], idx_v)

        zeros = jnp.zeros((_L,), jnp.float32)
        ones = jnp.ones((_L,), jnp.float32)

        @plsc.parallel_loop(0, n_pad // _L, unroll=8)
        def _zero(i):
            bins_v[pl.ds(pl.multiple_of(i * _L, _L), _L)] = zeros

        @plsc.parallel_loop(0, e_per_w // _L, unroll=8)
        def _scat(i):
            idx = idx_v[pl.ds(pl.multiple_of(i * _L, _L), _L)]
            plsc.addupdate_scatter(bins_v, [idx], ones)
        pltpu.sync_copy(bins_v, out_hbm.at[wid])

    return deg_kernel(dst)


def _dinv_call(deg_partials):
    def body(p_ref, o_ref):
        deg = 1.0 + jnp.sum(p_ref[...], axis=0, keepdims=True)
        o_ref[...] = 1.0 / jnp.sqrt(deg)

    n_pad = deg_partials.shape[1]
    return pl.pallas_call(
        body,
        out_shape=jax.ShapeDtypeStruct((1, n_pad), jnp.float32),
    )(deg_partials)


def _w_partials_call(src, dst, dinv, n_pad, num_workers, nc):
    e_total = src.shape[0]
    e_per_w = e_total // num_workers
    mesh = plsc.VectorSubcoreMesh(core_axis_name="c", subcore_axis_name="s")

    @functools.partial(
        pl.kernel,
        mesh=mesh,
        out_type=jax.ShapeDtypeStruct((num_workers, n_pad), jnp.float32),
        scratch_types=[
            pltpu.VMEM((e_per_w,), jnp.int32),
            pltpu.VMEM((e_per_w,), jnp.int32),
            pltpu.VMEM((n_pad,), jnp.float32),
            pltpu.VMEM((n_pad,), jnp.float32),
        ],
        compiler_params=pltpu.CompilerParams(needs_layout_passes=False),
    )
    def w_kernel(src_hbm, dst_hbm, dinv_hbm, out_hbm, src_v, dst_v, dinv_v, bins_v):
        wid = lax.axis_index("s") * nc + lax.axis_index("c")
        base = wid * e_per_w
        pltpu.sync_copy(src_hbm.at[pl.ds(base, e_per_w)], src_v)
        pltpu.sync_copy(dst_hbm.at[pl.ds(base, e_per_w)], dst_v)
        pltpu.sync_copy(dinv_hbm, dinv_v)

        is_w0 = wid == 0
        zeros = jnp.zeros((_L,), jnp.float32)

        @plsc.parallel_loop(0, n_pad // _L, unroll=8)
        def _init(i):
            sl = pl.ds(pl.multiple_of(i * _L, _L), _L)
            dv = dinv_v[sl]
            bins_v[sl] = jnp.where(is_w0, dv * dv, zeros)

        @plsc.parallel_loop(0, e_per_w // _L, unroll=8)
        def _scat(i):
            sl = pl.ds(pl.multiple_of(i * _L, _L), _L)
            s = src_v[sl]
            d = dst_v[sl]
            a = plsc.load_gather(dinv_v, [s])
            b = plsc.load_gather(dinv_v, [d])
            plsc.addupdate_scatter(bins_v, [s], a * b)
        pltpu.sync_copy(bins_v, out_hbm.at[wid])

    return w_kernel(src, dst, dinv.reshape(-1))


def _pool_lstm_call(xr, w_partials, W_gcn, b_gcn, W_ihT, W_hhT, b_ih, b_hh,
                    W_clsT, b_cls, Bsz, Tlen):
    BT, Nn, Fin = xr.shape
    H = W_hhT.shape[0]
    O = W_clsT.shape[1]
    num_w = w_partials.shape[0]
    CN = 1000
    grid = Nn // CN
    inv_n = 1.0 / Nn
    wp3 = w_partials[:, :Nn].reshape(num_w, grid, 1, CN)

    def body(x_ref, wp_ref, wg_ref, bg_ref, wih_ref, whh_ref, bih_ref,
             bhh_ref, wcls_ref, bcls_ref, o_ref, acc_ref):
        i = pl.program_id(0)

        @pl.when(i == 0)
        def _():
            acc_ref[...] = jnp.zeros_like(acc_ref)

        w_chunk = jnp.sum(wp_ref[..., 0, :], axis=0)
        x = x_ref[...]
        BTl, CNl, Fl = x.shape
        xl = jnp.dot(x.reshape(BTl * CNl, Fl), wg_ref[...],
                     preferred_element_type=jnp.float32)
        xl = xl.reshape(BTl, CNl, xl.shape[-1])
        acc_ref[...] += jnp.sum(xl * w_chunk[:, :, None], axis=1)

        @pl.when(i == grid - 1)
        def _():
            pooled = acc_ref[...] * inv_n + bg_ref[...]
            h = jnp.zeros((Bsz, H), jnp.float32)
            c = jnp.zeros((Bsz, H), jnp.float32)
            b_gates = bih_ref[...] + bhh_ref[...]
            for t in range(Tlen):
                xt = jnp.concatenate(
                    [pooled[b * Tlen + t:b * Tlen + t + 1] for b in range(Bsz)],
                    axis=0)
                gates = (jnp.dot(xt, wih_ref[...],
                                 preferred_element_type=jnp.float32)
                         + jnp.dot(h, whh_ref[...],
                                   preferred_element_type=jnp.float32)
                         + b_gates)
                i_g = jax.nn.sigmoid(gates[:, 0:H])
                f_g = jax.nn.sigmoid(gates[:, H:2 * H])
                g_g = jnp.tanh(gates[:, 2 * H:3 * H])
                o_g = jax.nn.sigmoid(gates[:, 3 * H:4 * H])
                c = f_g * c + i_g * g_g
                h = o_g * jnp.tanh(c)
            o_ref[...] = jnp.dot(h, wcls_ref[...],
                                 preferred_element_type=jnp.float32) + bcls_ref[...]

    full = lambda shape: pl.BlockSpec(shape, lambda i: tuple(0 for _ in shape))
    return pl.pallas_call(
        body,
        grid=(grid,),
        in_specs=[
            pl.BlockSpec((BT, CN, Fin), lambda i: (0, i, 0)),
            pl.BlockSpec((num_w, 1, 1, CN), lambda i: (0, i, 0, 0)),
            full(W_gcn.shape),
            full(b_gcn.shape),
            full(W_ihT.shape),
            full(W_hhT.shape),
            full(b_ih.shape),
            full(b_hh.shape),
            full(W_clsT.shape),
            full(b_cls.shape),
        ],
        out_specs=pl.BlockSpec((Bsz, O), lambda i: (0, 0)),
        out_shape=jax.ShapeDtypeStruct((Bsz, O), jnp.float32),
        scratch_shapes=[pltpu.VMEM((BT, Fin), jnp.float32)],
        compiler_params=pltpu.CompilerParams(
            dimension_semantics=("arbitrary",)),
    )(xr, wp3, W_gcn, b_gcn, W_ihT, W_hhT, b_ih, b_hh, W_clsT, b_cls)


def kernel(x_seq, edge_index, W_gcn, b_gcn, W_ih, W_hh, b_ih, b_hh, W_cls, b_cls):
    Bsz, Tlen, Nn, Fin = x_seq.shape
    info = plsc.get_sparse_core_info()
    nc, ns = info.num_cores, info.num_subcores
    num_workers = nc * ns

    n_pad = ((Nn + 127) // 128) * 128
    src = edge_index[0]
    dst = edge_index[1]

    deg_partials = _deg_partials_call(dst, n_pad, num_workers, nc)
    dinv = _dinv_call(deg_partials)
    w_partials = _w_partials_call(src, dst, dinv, n_pad, num_workers, nc)

    xr = x_seq.reshape(Bsz * Tlen, Nn, Fin)
    logits = _pool_lstm_call(
        xr, w_partials, W_gcn, b_gcn.reshape(1, -1), W_ih.T, W_hh.T,
        b_ih.reshape(1, -1), b_hh.reshape(1, -1), W_cls.T,
        b_cls.reshape(1, -1), Bsz, Tlen)
    return logits
